# speedup vs baseline: 10.7896x; 10.7896x over previous
import jax
import jax.numpy as jnp
from jax.experimental import pallas as pl
from jax.experimental.pallas import tpu as pltpu

EPS = 1e-5
LT = 256
HC = 1024


def _body(x_ref, a_ref, b_ref, owt_ref, ob_ref, n1_ref, n2_ref,
          w1t_ref, w1b_ref, w2t_ref, w2b_ref, w3t_ref, w3b_ref,
          o_ref, carry_ref):
    t = pl.program_id(1)

    @pl.when(t == 0)
    def _():
        carry_ref[...] = jnp.zeros_like(carry_ref)

    xb = x_ref[0]
    D = xb.shape[-1]
    H = w1b_ref.shape[-1]

    rms1 = jax.lax.rsqrt(jnp.mean(xb * xb, axis=-1, keepdims=True) + EPS)
    y = (xb * rms1) * (n1_ref[...] * b_ref[...])

    at = jnp.tanh(a_ref[...])
    row = jax.lax.broadcasted_iota(jnp.int32, (LT, 1), 0)
    y = y + jnp.where(row == 0, jnp.float32(1.0), jnp.float32(0.0)) * (at * carry_ref[...])
    p = at
    k = 1
    while k < LT:
        shifted = jnp.concatenate(
            [jnp.zeros((k, D), jnp.float32), y[: LT - k]], axis=0)
        y = y + p * shifted
        p = p * p
        k *= 2
    carry_ref[...] = y[LT - 1: LT]

    ssm = jnp.dot(y.astype(jnp.bfloat16), owt_ref[...],
                  preferred_element_type=jnp.float32)
    x1 = xb + ssm + ob_ref[...]

    rms2 = jax.lax.rsqrt(jnp.mean(x1 * x1, axis=-1, keepdims=True) + EPS)
    x1n = ((x1 * rms2) * n2_ref[...]).astype(jnp.bfloat16)

    o = x1
    for hc in range(H // HC):
        sl = slice(hc * HC, (hc + 1) * HC)
        u = jnp.dot(x1n, w1t_ref[:, sl], preferred_element_type=jnp.float32) \
            + w1b_ref[:, sl]
        g = jnp.dot(x1n, w2t_ref[:, sl], preferred_element_type=jnp.float32) \
            + w2b_ref[:, sl]
        h = (jax.nn.silu(g) * u).astype(jnp.bfloat16)
        o = o + jnp.dot(h, w3t_ref[sl, :], preferred_element_type=jnp.float32)
    o_ref[0] = o + w3b_ref[...]


def kernel(x, a, b, out_w, out_b, n1_w, n2_w,
           w1_w, w1_b, w2_w, w2_b, w3_w, w3_b):
    B, T, D = x.shape
    H = w1_w.shape[0]

    owt = out_w.T.astype(jnp.bfloat16)
    w1t = w1_w.T.astype(jnp.bfloat16)
    w2t = w2_w.T.astype(jnp.bfloat16)
    w3t = w3_w.T.astype(jnp.bfloat16)

    a2 = a.reshape(1, D)
    b2 = b.reshape(1, D)
    ob2 = out_b.reshape(1, D)
    n12 = n1_w.reshape(1, D)
    n22 = n2_w.reshape(1, D)
    w1b2 = w1_b.reshape(1, H)
    w2b2 = w2_b.reshape(1, H)
    w3b2 = w3_b.reshape(1, D)

    vec_spec_d = pl.BlockSpec((1, D), lambda i, j: (0, 0))
    vec_spec_h = pl.BlockSpec((1, H), lambda i, j: (0, 0))
    full = lambda shape: pl.BlockSpec(shape, lambda i, j: (0, 0))

    grid = (B, T // LT)
    out = pl.pallas_call(
        _body,
        grid=grid,
        in_specs=[
            pl.BlockSpec((1, LT, D), lambda i, j: (i, j, 0)),
            vec_spec_d,
            vec_spec_d,
            full((D, D)),
            vec_spec_d,
            vec_spec_d,
            vec_spec_d,
            full((D, H)),
            vec_spec_h,
            full((D, H)),
            vec_spec_h,
            full((H, D)),
            vec_spec_d,
        ],
        out_specs=pl.BlockSpec((1, LT, D), lambda i, j: (i, j, 0)),
        out_shape=jax.ShapeDtypeStruct((B, T, D), jnp.float32),
        scratch_shapes=[pltpu.VMEM((1, D), jnp.float32)],
        compiler_params=pltpu.CompilerParams(
            dimension_semantics=("parallel", "arbitrary"),
            vmem_limit_bytes=60000 * 1024,
        ),
        name="ssm_block_fused",
    )(x, a2, b2, owt, ob2, n12, n22, w1t, w1b2, w2t, w2b2, w3t, w3b2)
    return out
